# baseline (speedup 1.0000x reference)
import functools
import jax
import jax.numpy as jnp
from jax import lax
from jax.experimental import pallas as pl
from jax.experimental.pallas import tpu as pltpu
from jax.experimental.pallas import tpu_sc as plsc

_N = 65536
_K = 128
_M = 16
_NW = 32
_RPW = _N // _NW
_CH = 256


def _sc_body(x_hbm, w_hbm, o_hbm, wbuf, xbuf, obuf, sem):
    cid = lax.axis_index("c")
    sid = lax.axis_index("s")
    wid = sid * 2 + cid
    base = wid * _RPW
    pltpu.sync_copy(w_hbm, wbuf)

    for ch in range(_RPW // _CH):
        pltpu.async_copy(
            x_hbm.at[pl.ds(base + ch * _CH, _CH), :], xbuf, sem).wait()

        def row_body(i, _):
            acc = jnp.zeros((16,), jnp.float32)
            for c in range(_K // 16):
                xv = xbuf[i, pl.ds(c * 16, 16)]
                for j in range(16):
                    acc = acc + xv[j] * wbuf[c * 16 + j, :]
            obuf[i, :] = acc
            return _

        lax.fori_loop(0, _CH, row_body, 0)
        pltpu.sync_copy(obuf,
                        o_hbm.at[pl.ds(base + ch * _CH, _CH), :])


@functools.partial(
    pl.kernel,
    out_type=jax.ShapeDtypeStruct((_N, _M), jnp.float32),
    mesh=plsc.VectorSubcoreMesh(core_axis_name="c", subcore_axis_name="s"),
    scratch_types=[
        pltpu.VMEM((_K, _M), jnp.float32),
        pltpu.VMEM((_CH, _K), jnp.float32),
        pltpu.VMEM((_CH, _M), jnp.float32),
        pltpu.SemaphoreType.DMA,
    ],
)
def _sc_kernel(x_hbm, w_hbm, o_hbm, wbuf, xbuf, obuf, sem):
    _sc_body(x_hbm, w_hbm, o_hbm, wbuf, xbuf, obuf, sem)


def kernel(x, W):
    return _sc_kernel(x, W)


# speedup vs baseline: 25.2198x; 25.2198x over previous
import jax
import jax.numpy as jnp
from jax import lax
from jax.experimental import pallas as pl
from jax.experimental.pallas import tpu as pltpu

_CHUNK = 16384


def _mm_body(x_ref, w_ref, o_ref):
    o_ref[...] = lax.dot_general(
        w_ref[...], x_ref[...],
        (((0,), (1,)), ((), ())),
        preferred_element_type=jnp.float32,
    )


def kernel(x, W):
    n, k = x.shape
    m = W.shape[1]
    grid = n // _CHUNK
    out_t = pl.pallas_call(
        _mm_body,
        grid=(grid,),
        in_specs=[
            pl.BlockSpec((_CHUNK, k), lambda i: (i, 0)),
            pl.BlockSpec((k, m), lambda i: (0, 0)),
        ],
        out_specs=pl.BlockSpec((m, _CHUNK), lambda i: (0, i)),
        out_shape=jax.ShapeDtypeStruct((m, n), jnp.float32),
        compiler_params=pltpu.CompilerParams(
            dimension_semantics=("parallel",),
        ),
    )(x, W)
    return out_t.T


# speedup vs baseline: 25.2720x; 1.0021x over previous
import jax
import jax.numpy as jnp
from jax import lax
from jax.experimental import pallas as pl
from jax.experimental.pallas import tpu as pltpu

_CHUNK = 16384


def _mm_body(x_ref, w_ref, o_ref):
    o_ref[...] = lax.dot_general(
        w_ref[...], x_ref[...],
        (((0,), (1,)), ((), ())),
        preferred_element_type=jnp.float32,
    )


def kernel(x, W):
    n, k = x.shape
    m = W.shape[1]
    grid = n // _CHUNK
    out_t = pl.pallas_call(
        _mm_body,
        grid=(grid,),
        in_specs=[
            pl.BlockSpec((_CHUNK, k), lambda i: (i, 0)),
            pl.BlockSpec((k, m), lambda i: (0, 0)),
        ],
        out_specs=pl.BlockSpec((m, _CHUNK), lambda i: (0, i)),
        out_shape=jax.ShapeDtypeStruct((m, n), jnp.float32),
        compiler_params=pltpu.CompilerParams(
            dimension_semantics=("arbitrary",),
        ),
    )(x, W)
    return out_t.T
